# baseline (speedup 1.0000x reference)
import functools

import jax
import jax.numpy as jnp
from jax.experimental import pallas as pl
from jax.experimental.pallas import tpu as pltpu

NSAMPLES = 32
NPOINTS = 2048
K = 32
L = 1.0

TN = 64


def _dot(x, W):
    if W.shape[0] == 1:
        return x * W[0:1, :]
    return jnp.dot(x, W, preferred_element_type=jnp.float32)


def _pyr_fwd(Ws, bs, x):
    ys = []
    x = jnp.tanh(_dot(x, Ws[0]) + bs[0])
    ys.append(x)
    for W, b in zip(Ws[1:], bs[1:]):
        din, dout = W.shape
        y = jnp.tanh(_dot(x, W) + b)
        ys.append(y)
        if dout == 2 * din:
            x = jnp.concatenate([x, x], axis=-1) + y
        elif dout == din:
            x = x + y
        else:
            x = y
    return x, ys


def _pyr_bwd(Ws, ys, dx):
    for l in range(len(Ws) - 1, 0, -1):
        W = Ws[l]
        din, dout = W.shape
        y = ys[l]
        dz = dx * (1.0 - y * y)
        dxl = _dot(dz, W.T)
        if dout == 2 * din:
            dx = dx[:, :din] + dx[:, din:] + dxl
        elif dout == din:
            dx = dx + dxl
        else:
            dx = dxl
    y0 = ys[0]
    dz = dx * (1.0 - y0 * y0)
    return _dot(dz, Ws[0].T)


def _tc_body(nw, *refs):
    dx_ref, dy_ref, nb_ref = refs[:3]
    wrefs = refs[3:3 + nw]
    brefs = refs[3 + nw:3 + 2 * nw]
    en_ref, gx_ref, gy_ref = refs[3 + 2 * nw:]

    Wp = [wrefs[i][...] for i in range(5)]
    bp = [brefs[i][...] for i in range(5)]
    Wd = [wrefs[5 + i][...] for i in range(5)]
    bd = [brefs[5 + i][...] for i in range(5)]
    Wf = [wrefs[10 + i][...] for i in range(5)]
    bf = [brefs[10 + i][...] for i in range(5)]
    Wl = wrefs[15][...]
    bl = brefs[15][...]

    T = TN * K
    dx = dx_ref[0]
    dy = dy_ref[0]
    nb = nb_ref[0]

    dx = dx - L * jnp.round(dx * (1.0 / L))
    dy = dy - L * jnp.round(dy * (1.0 / L))
    r2 = dx * dx + dy * dy
    mask = (nb >= 0) & (r2 > 1e-12)
    safe_r2 = jnp.where(mask, r2, 1.0)
    rinv = jax.lax.rsqrt(safe_r2)
    fm = mask.astype(jnp.float32)

    s = rinv * fm
    cx = dx / safe_r2 * fm
    cy = dy / safe_r2 * fm

    P1, ys1 = _pyr_fwd(Wp, bp, s)
    P2, ys2 = _pyr_fwd(Wd, bd, jnp.concatenate([cx, cy], axis=-1))
    L1 = P1 * s
    L2 = P2 * s
    LL = jnp.concatenate([L1, L2], axis=-1)
    D = LL.reshape(TN, K, 64).sum(axis=1)
    F2, ysf = _pyr_fwd(Wf, bf, D)
    F = _dot(F2, Wl) + bl

    dF2 = jnp.broadcast_to(Wl.T, (TN, Wl.shape[0]))
    dD = _pyr_bwd(Wf, ysf, dF2)
    dLL = jnp.broadcast_to(dD.reshape(TN, 1, 64), (TN, K, 64)).reshape(T, 64)
    dL1 = dLL[:, :32]
    dL2 = dLL[:, 32:]
    ds = (jnp.sum(dL1 * P1, axis=-1, keepdims=True)
          + jnp.sum(dL2 * P2, axis=-1, keepdims=True))
    dP1 = dL1 * s
    dP2 = dL2 * s
    ds = ds + _pyr_bwd(Wp, ys1, dP1)
    dcxy = _pyr_bwd(Wd, ys2, dP2)

    u0 = ds
    ux = dcxy[:, 0:1]
    uy = dcxy[:, 1:2]

    inv_r2 = 1.0 / safe_r2
    inv_r4 = inv_r2 * inv_r2
    rinv3 = rinv * inv_r2
    gx = u0 * (-dx * rinv3) + ux * (safe_r2 - 2.0 * dx * dx) * inv_r4 \
        + uy * (-2.0 * dx * dy * inv_r4)
    gy = u0 * (-dy * rinv3) + uy * (safe_r2 - 2.0 * dy * dy) * inv_r4 \
        + ux * (-2.0 * dx * dy * inv_r4)
    gx = gx * fm
    gy = gy * fm

    gx_ref[0] = gx
    gy_ref[0] = gy

    e_tile = jnp.sum(F)

    @pl.when(pl.program_id(1) == 0)
    def _():
        en_ref[0] = jnp.zeros((8, 128), jnp.float32)

    en_ref[0] += jnp.broadcast_to(e_tile, (8, 128))


def _tc_call(dxf, dyf, nbf, wlist, blist):
    nw = len(wlist)
    T = TN * K
    grid = (NSAMPLES, NPOINTS // TN)

    def rep(shape):
        return pl.BlockSpec(shape, lambda i, j: (0,) * len(shape))

    pair_spec = pl.BlockSpec((1, T, 1), lambda i, j: (i, j, 0))
    in_specs = (
        [pair_spec, pair_spec, pair_spec]
        + [rep(w.shape) for w in wlist]
        + [rep(b.shape) for b in blist]
    )
    out_specs = [
        pl.BlockSpec((1, 8, 128), lambda i, j: (i, 0, 0)),
        pair_spec,
        pair_spec,
    ]
    out_shapes = [
        jax.ShapeDtypeStruct((NSAMPLES, 8, 128), jnp.float32),
        jax.ShapeDtypeStruct((NSAMPLES, NPOINTS * K, 1), jnp.float32),
        jax.ShapeDtypeStruct((NSAMPLES, NPOINTS * K, 1), jnp.float32),
    ]
    return pl.pallas_call(
        functools.partial(_tc_body, nw),
        grid=grid,
        in_specs=in_specs,
        out_specs=out_specs,
        out_shape=out_shapes,
    )(dxf, dyf, nbf, *wlist, *blist)


def kernel(inputs, params, neighList):
    R = inputs.astype(jnp.float32)
    nb = neighList.astype(jnp.int32)
    safe = jnp.where(nb >= 0, nb, 0)

    Rj = jax.vmap(lambda r, idx: r[idx])(R, safe)
    dxf = (Rj[..., 0] - R[:, :, None, 0]).reshape(NSAMPLES, NPOINTS * K, 1)
    dyf = (Rj[..., 1] - R[:, :, None, 1]).reshape(NSAMPLES, NPOINTS * K, 1)
    nbf = nb.reshape(NSAMPLES, NPOINTS * K, 1)

    wlist, blist = [], []
    for group in ("pyr", "pyr_dir", "fit"):
        for W, b in params[group]:
            wlist.append(W)
            blist.append(b.reshape(1, -1))
    Wl, bl = params["lin"]
    wlist.append(Wl)
    blist.append(bl.reshape(1, -1))

    en, gx, gy = _tc_call(dxf, dyf, nbf, wlist, blist)
    en = en[:, 0, 0:1]

    gsx = gx.reshape(NSAMPLES, NPOINTS, K).sum(axis=-1)
    gsy = gy.reshape(NSAMPLES, NPOINTS, K).sum(axis=-1)
    scx = jax.vmap(lambda g, idx: jnp.zeros(NPOINTS).at[idx].add(g))(
        gx.reshape(NSAMPLES, -1), safe.reshape(NSAMPLES, -1))
    scy = jax.vmap(lambda g, idx: jnp.zeros(NPOINTS).at[idx].add(g))(
        gy.reshape(NSAMPLES, -1), safe.reshape(NSAMPLES, -1))

    Fx = gsx - scx
    Fy = gsy - scy
    Forces = jnp.stack([Fx, Fy], axis=-1)
    return (en, Forces)


# speedup vs baseline: 3.1204x; 3.1204x over previous
import functools

import jax
import jax.numpy as jnp
from jax import lax
from jax.experimental import pallas as pl
from jax.experimental.pallas import tpu as pltpu
from jax.experimental.pallas import tpu_sc as plsc

NSAMPLES = 32
NPOINTS = 2048
K = 32
L = 1.0
NK = NPOINTS * K

TN = 64

SC_CH = 8192


def _sc_mesh():
    return plsc.VectorSubcoreMesh(core_axis_name="c", subcore_axis_name="s")


def _wid():
    return lax.axis_index("s") * 2 + lax.axis_index("c")


def _sc_gather(rx, ry, nbf):
    import functools

    @functools.partial(
        pl.kernel, mesh=_sc_mesh(),
        compiler_params=pltpu.CompilerParams(needs_layout_passes=False),
        out_type=[jax.ShapeDtypeStruct((NSAMPLES, NK), jnp.float32),
                  jax.ShapeDtypeStruct((NSAMPLES, NK), jnp.float32)],
        scratch_types=[pltpu.VMEM((NPOINTS,), jnp.float32),
                       pltpu.VMEM((NPOINTS,), jnp.float32),
                       pltpu.VMEM((SC_CH,), jnp.int32),
                       pltpu.VMEM((SC_CH,), jnp.float32),
                       pltpu.VMEM((SC_CH,), jnp.float32)],
    )
    def k(rx_hbm, ry_hbm, nb_hbm, dx_hbm, dy_hbm, rx_v, ry_v, idx_v, dx_v, dy_v):
        w = _wid()
        pltpu.sync_copy(rx_hbm.at[w], rx_v)
        pltpu.sync_copy(ry_hbm.at[w], ry_v)
        for c in range(NK // SC_CH):
            pltpu.sync_copy(nb_hbm.at[w, pl.ds(c * SC_CH, SC_CH)], idx_v)

            def body(i, _):
                idx = idx_v[pl.ds(i * 16, 16)]
                safe = jnp.maximum(idx, 0)
                pair = c * SC_CH + i * 16 + lax.iota(jnp.int32, 16)
                own = lax.shift_right_logical(pair, 5)
                rjx = plsc.load_gather(rx_v, [safe])
                rjy = plsc.load_gather(ry_v, [safe])
                rix = plsc.load_gather(rx_v, [own])
                riy = plsc.load_gather(ry_v, [own])
                dx_v[pl.ds(i * 16, 16)] = rjx - rix
                dy_v[pl.ds(i * 16, 16)] = rjy - riy
                return _

            lax.fori_loop(0, SC_CH // 16, body, None)
            pltpu.sync_copy(dx_v, dx_hbm.at[w, pl.ds(c * SC_CH, SC_CH)])
            pltpu.sync_copy(dy_v, dy_hbm.at[w, pl.ds(c * SC_CH, SC_CH)])

    return k(rx, ry, nbf)


def _sc_scatter(gxf, gyf, nbf):
    import functools

    @functools.partial(
        pl.kernel, mesh=_sc_mesh(),
        compiler_params=pltpu.CompilerParams(needs_layout_passes=False),
        out_type=[jax.ShapeDtypeStruct((NSAMPLES, NPOINTS), jnp.float32),
                  jax.ShapeDtypeStruct((NSAMPLES, NPOINTS), jnp.float32),
                  jax.ShapeDtypeStruct((NSAMPLES, NPOINTS), jnp.float32),
                  jax.ShapeDtypeStruct((NSAMPLES, NPOINTS), jnp.float32)],
        scratch_types=[pltpu.VMEM((NPOINTS,), jnp.float32),
                       pltpu.VMEM((NPOINTS,), jnp.float32),
                       pltpu.VMEM((NPOINTS,), jnp.float32),
                       pltpu.VMEM((NPOINTS,), jnp.float32),
                       pltpu.VMEM((SC_CH,), jnp.int32),
                       pltpu.VMEM((SC_CH,), jnp.float32),
                       pltpu.VMEM((SC_CH,), jnp.float32)],
    )
    def k(gx_hbm, gy_hbm, nb_hbm, ox_hbm, oy_hbm, sx_hbm, sy_hbm,
          ox_v, oy_v, sx_v, sy_v, idx_v, gx_v, gy_v):
        w = _wid()
        zeros = jnp.zeros((16,), jnp.float32)

        def zbody(i, _):
            ox_v[pl.ds(i * 16, 16)] = zeros
            oy_v[pl.ds(i * 16, 16)] = zeros
            sx_v[pl.ds(i * 16, 16)] = zeros
            sy_v[pl.ds(i * 16, 16)] = zeros
            return _

        lax.fori_loop(0, NPOINTS // 16, zbody, None)
        for c in range(NK // SC_CH):
            pltpu.sync_copy(nb_hbm.at[w, pl.ds(c * SC_CH, SC_CH)], idx_v)
            pltpu.sync_copy(gx_hbm.at[w, pl.ds(c * SC_CH, SC_CH)], gx_v)
            pltpu.sync_copy(gy_hbm.at[w, pl.ds(c * SC_CH, SC_CH)], gy_v)

            def body(i, _):
                idx = idx_v[pl.ds(i * 16, 16)]
                safe = jnp.maximum(idx, 0)
                pair = c * SC_CH + i * 16 + lax.iota(jnp.int32, 16)
                own = lax.shift_right_logical(pair, 5)
                gx = gx_v[pl.ds(i * 16, 16)]
                gy = gy_v[pl.ds(i * 16, 16)]
                plsc.addupdate_scatter(sx_v, [safe], gx)
                plsc.addupdate_scatter(sy_v, [safe], gy)
                plsc.addupdate_scatter(ox_v, [own], gx)
                plsc.addupdate_scatter(oy_v, [own], gy)
                return _

            lax.fori_loop(0, SC_CH // 16, body, None)
        pltpu.sync_copy(ox_v, ox_hbm.at[w])
        pltpu.sync_copy(oy_v, oy_hbm.at[w])
        pltpu.sync_copy(sx_v, sx_hbm.at[w])
        pltpu.sync_copy(sy_v, sy_hbm.at[w])

    return k(gxf, gyf, nbf)


def _dot(x, W):
    if W.shape[0] == 1:
        return x * W[0:1, :]
    return jnp.dot(x, W, preferred_element_type=jnp.float32)


def _pyr_fwd(Ws, bs, x):
    ys = []
    x = jnp.tanh(_dot(x, Ws[0]) + bs[0])
    ys.append(x)
    for W, b in zip(Ws[1:], bs[1:]):
        din, dout = W.shape
        y = jnp.tanh(_dot(x, W) + b)
        ys.append(y)
        if dout == 2 * din:
            x = jnp.concatenate([x, x], axis=-1) + y
        elif dout == din:
            x = x + y
        else:
            x = y
    return x, ys


def _pyr_bwd(Ws, ys, dx):
    for l in range(len(Ws) - 1, 0, -1):
        W = Ws[l]
        din, dout = W.shape
        y = ys[l]
        dz = dx * (1.0 - y * y)
        dxl = _dot(dz, W.T)
        if dout == 2 * din:
            dx = dx[:, :din] + dx[:, din:] + dxl
        elif dout == din:
            dx = dx + dxl
        else:
            dx = dxl
    y0 = ys[0]
    dz = dx * (1.0 - y0 * y0)
    return _dot(dz, Ws[0].T)


def _tc_body(nw, *refs):
    dx_ref, dy_ref, nb_ref = refs[:3]
    wrefs = refs[3:3 + nw]
    brefs = refs[3 + nw:3 + 2 * nw]
    en_ref, gx_ref, gy_ref = refs[3 + 2 * nw:]

    Wp = [wrefs[i][...] for i in range(5)]
    bp = [brefs[i][...] for i in range(5)]
    Wd = [wrefs[5 + i][...] for i in range(5)]
    bd = [brefs[5 + i][...] for i in range(5)]
    Wf = [wrefs[10 + i][...] for i in range(5)]
    bf = [brefs[10 + i][...] for i in range(5)]
    Wl = wrefs[15][...]
    bl = brefs[15][...]

    T = TN * K
    dx = dx_ref[0]
    dy = dy_ref[0]
    nb = nb_ref[0]

    dx = dx - L * jnp.round(dx * (1.0 / L))
    dy = dy - L * jnp.round(dy * (1.0 / L))
    r2 = dx * dx + dy * dy
    mask = (nb >= 0) & (r2 > 1e-12)
    safe_r2 = jnp.where(mask, r2, 1.0)
    rinv = jax.lax.rsqrt(safe_r2)
    fm = mask.astype(jnp.float32)

    s = rinv * fm
    cx = dx / safe_r2 * fm
    cy = dy / safe_r2 * fm

    P1, ys1 = _pyr_fwd(Wp, bp, s)
    P2, ys2 = _pyr_fwd(Wd, bd, jnp.concatenate([cx, cy], axis=-1))
    L1 = P1 * s
    L2 = P2 * s
    LL = jnp.concatenate([L1, L2], axis=-1)
    D = LL.reshape(TN, K, 64).sum(axis=1)
    F2, ysf = _pyr_fwd(Wf, bf, D)
    F = _dot(F2, Wl) + bl

    dF2 = jnp.broadcast_to(Wl.T, (TN, Wl.shape[0]))
    dD = _pyr_bwd(Wf, ysf, dF2)
    dLL = jnp.broadcast_to(dD.reshape(TN, 1, 64), (TN, K, 64)).reshape(T, 64)
    dL1 = dLL[:, :32]
    dL2 = dLL[:, 32:]
    ds = (jnp.sum(dL1 * P1, axis=-1, keepdims=True)
          + jnp.sum(dL2 * P2, axis=-1, keepdims=True))
    dP1 = dL1 * s
    dP2 = dL2 * s
    ds = ds + _pyr_bwd(Wp, ys1, dP1)
    dcxy = _pyr_bwd(Wd, ys2, dP2)

    u0 = ds
    ux = dcxy[:, 0:1]
    uy = dcxy[:, 1:2]

    inv_r2 = 1.0 / safe_r2
    inv_r4 = inv_r2 * inv_r2
    rinv3 = rinv * inv_r2
    gx = u0 * (-dx * rinv3) + ux * (safe_r2 - 2.0 * dx * dx) * inv_r4 \
        + uy * (-2.0 * dx * dy * inv_r4)
    gy = u0 * (-dy * rinv3) + uy * (safe_r2 - 2.0 * dy * dy) * inv_r4 \
        + ux * (-2.0 * dx * dy * inv_r4)
    gx = gx * fm
    gy = gy * fm

    gx_ref[0] = gx
    gy_ref[0] = gy

    e_tile = jnp.sum(F)

    @pl.when(pl.program_id(1) == 0)
    def _():
        en_ref[0] = jnp.zeros((8, 128), jnp.float32)

    en_ref[0] += jnp.broadcast_to(e_tile, (8, 128))


def _tc_call(dxf, dyf, nbf, wlist, blist):
    nw = len(wlist)
    T = TN * K
    grid = (NSAMPLES, NPOINTS // TN)

    def rep(shape):
        return pl.BlockSpec(shape, lambda i, j: (0,) * len(shape))

    pair_spec = pl.BlockSpec((1, T, 1), lambda i, j: (i, j, 0))
    in_specs = (
        [pair_spec, pair_spec, pair_spec]
        + [rep(w.shape) for w in wlist]
        + [rep(b.shape) for b in blist]
    )
    out_specs = [
        pl.BlockSpec((1, 8, 128), lambda i, j: (i, 0, 0)),
        pair_spec,
        pair_spec,
    ]
    out_shapes = [
        jax.ShapeDtypeStruct((NSAMPLES, 8, 128), jnp.float32),
        jax.ShapeDtypeStruct((NSAMPLES, NPOINTS * K, 1), jnp.float32),
        jax.ShapeDtypeStruct((NSAMPLES, NPOINTS * K, 1), jnp.float32),
    ]
    return pl.pallas_call(
        functools.partial(_tc_body, nw),
        grid=grid,
        in_specs=in_specs,
        out_specs=out_specs,
        out_shape=out_shapes,
    )(dxf, dyf, nbf, *wlist, *blist)


def kernel(inputs, params, neighList):
    R = inputs.astype(jnp.float32)
    nb = neighList.astype(jnp.int32)

    nbf2 = nb.reshape(NSAMPLES, NK)
    dxf2, dyf2 = _sc_gather(R[..., 0], R[..., 1], nbf2)
    dxf = dxf2.reshape(NSAMPLES, NK, 1)
    dyf = dyf2.reshape(NSAMPLES, NK, 1)
    nbf = nbf2.reshape(NSAMPLES, NK, 1)

    wlist, blist = [], []
    for group in ("pyr", "pyr_dir", "fit"):
        for W, b in params[group]:
            wlist.append(W)
            blist.append(b.reshape(1, -1))
    Wl, bl = params["lin"]
    wlist.append(Wl)
    blist.append(bl.reshape(1, -1))

    en, gx, gy = _tc_call(dxf, dyf, nbf, wlist, blist)
    en = en[:, 0, 0:1]

    ox, oy, sx, sy = _sc_scatter(gx.reshape(NSAMPLES, NK),
                                 gy.reshape(NSAMPLES, NK), nbf2)
    Forces = jnp.stack([ox - sx, oy - sy], axis=-1)
    return (en, Forces)


# speedup vs baseline: 3.2453x; 1.0400x over previous
import functools

import jax
import jax.numpy as jnp
from jax import lax
from jax.experimental import pallas as pl
from jax.experimental.pallas import tpu as pltpu
from jax.experimental.pallas import tpu_sc as plsc

NSAMPLES = 32
NPOINTS = 2048
K = 32
L = 1.0
NK = NPOINTS * K

TN = 128

SC_CH = 8192


def _sc_mesh():
    return plsc.VectorSubcoreMesh(core_axis_name="c", subcore_axis_name="s")


def _wid():
    return lax.axis_index("s") * 2 + lax.axis_index("c")


def _sc_gather(rx, ry, nbf):
    import functools

    @functools.partial(
        pl.kernel, mesh=_sc_mesh(),
        compiler_params=pltpu.CompilerParams(needs_layout_passes=False),
        out_type=[jax.ShapeDtypeStruct((NSAMPLES, NK), jnp.float32),
                  jax.ShapeDtypeStruct((NSAMPLES, NK), jnp.float32)],
        scratch_types=[pltpu.VMEM((NPOINTS,), jnp.float32),
                       pltpu.VMEM((NPOINTS,), jnp.float32),
                       pltpu.VMEM((SC_CH,), jnp.int32),
                       pltpu.VMEM((SC_CH,), jnp.float32),
                       pltpu.VMEM((SC_CH,), jnp.float32)],
    )
    def k(rx_hbm, ry_hbm, nb_hbm, dx_hbm, dy_hbm, rx_v, ry_v, idx_v, dx_v, dy_v):
        w = _wid()
        pltpu.sync_copy(rx_hbm.at[w], rx_v)
        pltpu.sync_copy(ry_hbm.at[w], ry_v)
        for c in range(NK // SC_CH):
            pltpu.sync_copy(nb_hbm.at[w, pl.ds(c * SC_CH, SC_CH)], idx_v)

            def body(i, _):
                idx = idx_v[pl.ds(i * 16, 16)]
                safe = jnp.maximum(idx, 0)
                pair = c * SC_CH + i * 16 + lax.iota(jnp.int32, 16)
                own = lax.shift_right_logical(pair, 5)
                rjx = plsc.load_gather(rx_v, [safe])
                rjy = plsc.load_gather(ry_v, [safe])
                rix = plsc.load_gather(rx_v, [own])
                riy = plsc.load_gather(ry_v, [own])
                dx_v[pl.ds(i * 16, 16)] = rjx - rix
                dy_v[pl.ds(i * 16, 16)] = rjy - riy
                return _

            lax.fori_loop(0, SC_CH // 16, body, None)
            pltpu.sync_copy(dx_v, dx_hbm.at[w, pl.ds(c * SC_CH, SC_CH)])
            pltpu.sync_copy(dy_v, dy_hbm.at[w, pl.ds(c * SC_CH, SC_CH)])

    return k(rx, ry, nbf)


def _sc_scatter(gxf, gyf, nbf):
    import functools

    @functools.partial(
        pl.kernel, mesh=_sc_mesh(),
        compiler_params=pltpu.CompilerParams(needs_layout_passes=False),
        out_type=[jax.ShapeDtypeStruct((NSAMPLES, NPOINTS), jnp.float32),
                  jax.ShapeDtypeStruct((NSAMPLES, NPOINTS), jnp.float32),
                  jax.ShapeDtypeStruct((NSAMPLES, NPOINTS), jnp.float32),
                  jax.ShapeDtypeStruct((NSAMPLES, NPOINTS), jnp.float32)],
        scratch_types=[pltpu.VMEM((NPOINTS,), jnp.float32),
                       pltpu.VMEM((NPOINTS,), jnp.float32),
                       pltpu.VMEM((NPOINTS,), jnp.float32),
                       pltpu.VMEM((NPOINTS,), jnp.float32),
                       pltpu.VMEM((SC_CH,), jnp.int32),
                       pltpu.VMEM((SC_CH,), jnp.float32),
                       pltpu.VMEM((SC_CH,), jnp.float32)],
    )
    def k(gx_hbm, gy_hbm, nb_hbm, ox_hbm, oy_hbm, sx_hbm, sy_hbm,
          ox_v, oy_v, sx_v, sy_v, idx_v, gx_v, gy_v):
        w = _wid()
        zeros = jnp.zeros((16,), jnp.float32)

        def zbody(i, _):
            ox_v[pl.ds(i * 16, 16)] = zeros
            oy_v[pl.ds(i * 16, 16)] = zeros
            sx_v[pl.ds(i * 16, 16)] = zeros
            sy_v[pl.ds(i * 16, 16)] = zeros
            return _

        lax.fori_loop(0, NPOINTS // 16, zbody, None)
        for c in range(NK // SC_CH):
            pltpu.sync_copy(nb_hbm.at[w, pl.ds(c * SC_CH, SC_CH)], idx_v)
            pltpu.sync_copy(gx_hbm.at[w, pl.ds(c * SC_CH, SC_CH)], gx_v)
            pltpu.sync_copy(gy_hbm.at[w, pl.ds(c * SC_CH, SC_CH)], gy_v)

            def body(i, _):
                idx = idx_v[pl.ds(i * 16, 16)]
                safe = jnp.maximum(idx, 0)
                pair = c * SC_CH + i * 16 + lax.iota(jnp.int32, 16)
                own = lax.shift_right_logical(pair, 5)
                gx = gx_v[pl.ds(i * 16, 16)]
                gy = gy_v[pl.ds(i * 16, 16)]
                plsc.addupdate_scatter(sx_v, [safe], gx)
                plsc.addupdate_scatter(sy_v, [safe], gy)
                plsc.addupdate_scatter(ox_v, [own], gx)
                plsc.addupdate_scatter(oy_v, [own], gy)
                return _

            lax.fori_loop(0, SC_CH // 16, body, None)
        pltpu.sync_copy(ox_v, ox_hbm.at[w])
        pltpu.sync_copy(oy_v, oy_hbm.at[w])
        pltpu.sync_copy(sx_v, sx_hbm.at[w])
        pltpu.sync_copy(sy_v, sy_hbm.at[w])

    return k(gxf, gyf, nbf)


def _dot(x, W):
    if W.shape[0] == 1:
        return x * W[0:1, :]
    return jnp.dot(x, W, preferred_element_type=jnp.float32)


def _pyr_fwd(Ws, bs, x):
    ys = []
    x = jnp.tanh(_dot(x, Ws[0]) + bs[0])
    ys.append(x)
    for W, b in zip(Ws[1:], bs[1:]):
        din, dout = W.shape
        y = jnp.tanh(_dot(x, W) + b)
        ys.append(y)
        if dout == 2 * din:
            x = jnp.concatenate([x, x], axis=-1) + y
        elif dout == din:
            x = x + y
        else:
            x = y
    return x, ys


def _pyr_bwd(Ws, ys, dx):
    for l in range(len(Ws) - 1, 0, -1):
        W = Ws[l]
        din, dout = W.shape
        y = ys[l]
        dz = dx * (1.0 - y * y)
        dxl = _dot(dz, W.T)
        if dout == 2 * din:
            dx = dx[:, :din] + dx[:, din:] + dxl
        elif dout == din:
            dx = dx + dxl
        else:
            dx = dxl
    y0 = ys[0]
    dz = dx * (1.0 - y0 * y0)
    return _dot(dz, Ws[0].T)


def _tc_body(nw, *refs):
    dx_ref, dy_ref, nb_ref = refs[:3]
    wrefs = refs[3:3 + nw]
    brefs = refs[3 + nw:3 + 2 * nw]
    en_ref, gx_ref, gy_ref = refs[3 + 2 * nw:]

    Wp = [wrefs[i][...] for i in range(5)]
    bp = [brefs[i][...] for i in range(5)]
    Wd = [wrefs[5 + i][...] for i in range(5)]
    bd = [brefs[5 + i][...] for i in range(5)]
    Wf = [wrefs[10 + i][...] for i in range(5)]
    bf = [brefs[10 + i][...] for i in range(5)]
    Wl = wrefs[15][...]
    bl = brefs[15][...]

    T = TN * K
    dx = dx_ref[0]
    dy = dy_ref[0]
    nb = nb_ref[0]

    dx = dx - L * jnp.round(dx * (1.0 / L))
    dy = dy - L * jnp.round(dy * (1.0 / L))
    r2 = dx * dx + dy * dy
    mask = (nb >= 0) & (r2 > 1e-12)
    safe_r2 = jnp.where(mask, r2, 1.0)
    rinv = jax.lax.rsqrt(safe_r2)
    fm = mask.astype(jnp.float32)

    s = rinv * fm
    cx = dx / safe_r2 * fm
    cy = dy / safe_r2 * fm

    P1, ys1 = _pyr_fwd(Wp, bp, s)
    P2, ys2 = _pyr_fwd(Wd, bd, jnp.concatenate([cx, cy], axis=-1))
    L1 = P1 * s
    L2 = P2 * s
    LL = jnp.concatenate([L1, L2], axis=-1)
    D = LL.reshape(TN, K, 64).sum(axis=1)
    F2, ysf = _pyr_fwd(Wf, bf, D)
    F = _dot(F2, Wl) + bl

    dF2 = jnp.broadcast_to(Wl.T, (TN, Wl.shape[0]))
    dD = _pyr_bwd(Wf, ysf, dF2)
    dLL = jnp.broadcast_to(dD.reshape(TN, 1, 64), (TN, K, 64)).reshape(T, 64)
    dL1 = dLL[:, :32]
    dL2 = dLL[:, 32:]
    ds = (jnp.sum(dL1 * P1, axis=-1, keepdims=True)
          + jnp.sum(dL2 * P2, axis=-1, keepdims=True))
    dP1 = dL1 * s
    dP2 = dL2 * s
    ds = ds + _pyr_bwd(Wp, ys1, dP1)
    dcxy = _pyr_bwd(Wd, ys2, dP2)

    u0 = ds
    ux = dcxy[:, 0:1]
    uy = dcxy[:, 1:2]

    inv_r2 = 1.0 / safe_r2
    inv_r4 = inv_r2 * inv_r2
    rinv3 = rinv * inv_r2
    gx = u0 * (-dx * rinv3) + ux * (safe_r2 - 2.0 * dx * dx) * inv_r4 \
        + uy * (-2.0 * dx * dy * inv_r4)
    gy = u0 * (-dy * rinv3) + uy * (safe_r2 - 2.0 * dy * dy) * inv_r4 \
        + ux * (-2.0 * dx * dy * inv_r4)
    gx = gx * fm
    gy = gy * fm

    gx_ref[0] = gx
    gy_ref[0] = gy

    e_tile = jnp.sum(F)

    @pl.when(pl.program_id(1) == 0)
    def _():
        en_ref[0] = jnp.zeros((8, 128), jnp.float32)

    en_ref[0] += jnp.broadcast_to(e_tile, (8, 128))


def _tc_call(dxf, dyf, nbf, wlist, blist):
    nw = len(wlist)
    T = TN * K
    grid = (NSAMPLES, NPOINTS // TN)

    def rep(shape):
        return pl.BlockSpec(shape, lambda i, j: (0,) * len(shape))

    pair_spec = pl.BlockSpec((1, T, 1), lambda i, j: (i, j, 0))
    in_specs = (
        [pair_spec, pair_spec, pair_spec]
        + [rep(w.shape) for w in wlist]
        + [rep(b.shape) for b in blist]
    )
    out_specs = [
        pl.BlockSpec((1, 8, 128), lambda i, j: (i, 0, 0)),
        pair_spec,
        pair_spec,
    ]
    out_shapes = [
        jax.ShapeDtypeStruct((NSAMPLES, 8, 128), jnp.float32),
        jax.ShapeDtypeStruct((NSAMPLES, NPOINTS * K, 1), jnp.float32),
        jax.ShapeDtypeStruct((NSAMPLES, NPOINTS * K, 1), jnp.float32),
    ]
    return pl.pallas_call(
        functools.partial(_tc_body, nw),
        grid=grid,
        in_specs=in_specs,
        out_specs=out_specs,
        out_shape=out_shapes,
        compiler_params=pltpu.CompilerParams(vmem_limit_bytes=110 * 1024 * 1024),
    )(dxf, dyf, nbf, *wlist, *blist)


def kernel(inputs, params, neighList):
    R = inputs.astype(jnp.float32)
    nb = neighList.astype(jnp.int32)

    nbf2 = nb.reshape(NSAMPLES, NK)
    dxf2, dyf2 = _sc_gather(R[..., 0], R[..., 1], nbf2)
    dxf = dxf2.reshape(NSAMPLES, NK, 1)
    dyf = dyf2.reshape(NSAMPLES, NK, 1)
    nbf = nbf2.reshape(NSAMPLES, NK, 1)

    wlist, blist = [], []
    for group in ("pyr", "pyr_dir", "fit"):
        for W, b in params[group]:
            wlist.append(W)
            blist.append(b.reshape(1, -1))
    Wl, bl = params["lin"]
    wlist.append(Wl)
    blist.append(bl.reshape(1, -1))

    en, gx, gy = _tc_call(dxf, dyf, nbf, wlist, blist)
    en = en[:, 0, 0:1]

    ox, oy, sx, sy = _sc_scatter(gx.reshape(NSAMPLES, NK),
                                 gy.reshape(NSAMPLES, NK), nbf2)
    Forces = jnp.stack([ox - sx, oy - sy], axis=-1)
    return (en, Forces)
